# baseline (speedup 1.0000x reference)
import functools

import jax
import jax.numpy as jnp
from jax import lax
from jax.experimental import pallas as pl
from jax.experimental.pallas import tpu as pltpu
from jax.experimental.pallas import tpu_sc as plsc


_CB = 32
_NBUF = 3


def _sc_body(w_hbm, o_hbm, bufs, in_sems, out_sems, *, bsz, dim, cb,
             rows_per_w, nchunks, nbuf, num_subcores):
    c = lax.axis_index("c")
    s = lax.axis_index("s")
    wid = c * num_subcores + s
    pair = wid // 2
    par = wid % 2
    base = pair * rows_per_w

    def in_copy(k):
        sl = k % nbuf
        return pltpu.make_async_copy(
            w_hbm.at[pl.ds(base + k * cb, cb), par, :],
            bufs.at[sl],
            in_sems.at[sl])

    def out_copy(k, b):
        sl = k % nbuf
        return pltpu.make_async_copy(
            bufs.at[sl],
            o_hbm.at[pl.ds(base + k * cb, cb), par, b, :],
            out_sems.at[sl, b])

    for k in range(min(nbuf, nchunks)):
        in_copy(k).start()
    for k in range(nchunks):
        in_copy(k).wait()
        for b in range(bsz):
            out_copy(k, b).start()
        j = k - (nbuf - 1)
        if j >= 0:
            for b in range(bsz):
                out_copy(j, b).wait()
            if j + nbuf < nchunks:
                in_copy(j + nbuf).start()
    for j in range(max(0, nchunks - nbuf + 1), nchunks):
        for b in range(bsz):
            out_copy(j, b).wait()


def kernel(input, weights):
    seq_len, bsz = input.shape
    init_size, dim = weights.shape
    info = plsc.get_sparse_core_info()
    nw = info.num_cores * info.num_subcores
    rows_per_w = seq_len // nw
    cb = _CB if rows_per_w % _CB == 0 else rows_per_w
    nchunks = rows_per_w // cb
    nbuf = min(_NBUF, nchunks)
    mesh = plsc.VectorSubcoreMesh(core_axis_name="c", subcore_axis_name="s")
    body = functools.partial(
        _sc_body, bsz=bsz, dim=dim, cb=cb, rows_per_w=rows_per_w,
        nchunks=nchunks, nbuf=nbuf, num_subcores=info.num_subcores)
    w2 = weights[:seq_len].reshape(seq_len // 2, 2, dim)
    out = pl.kernel(
        body,
        out_type=jax.ShapeDtypeStruct((seq_len // 2, 2, bsz, dim),
                                      weights.dtype),
        mesh=mesh,
        scratch_types=[
            pltpu.VMEM((nbuf, cb, dim), weights.dtype),
            pltpu.SemaphoreType.DMA((nbuf,)),
            pltpu.SemaphoreType.DMA((nbuf, bsz)),
        ],
    )(w2)
    return out.reshape(seq_len, bsz, dim)


# speedup vs baseline: 1.4422x; 1.4422x over previous
import functools

import jax
import jax.numpy as jnp
from jax import lax
from jax.experimental import pallas as pl
from jax.experimental.pallas import tpu as pltpu
from jax.experimental.pallas import tpu_sc as plsc


_CB = 32
_NBUF = 3


def _sc_body(w_hbm, o_hbm, bufs, in_sems, out_sems, *, bsz, dim, cb,
             rows_per_w, nchunks, nbuf, num_subcores):
    c = lax.axis_index("c")
    s = lax.axis_index("s")
    wid = c * num_subcores + s
    base = wid * rows_per_w

    def in_copy(k):
        sl = k % nbuf
        return pltpu.make_async_copy(
            w_hbm.at[pl.ds(base + k * cb, cb), :], bufs.at[sl],
            in_sems.at[sl])

    def out_copy(k, b):
        sl = k % nbuf
        return pltpu.make_async_copy(
            bufs.at[sl],
            o_hbm.at[pl.ds(base + k * cb, cb), b, :],
            out_sems.at[sl, b])

    for k in range(min(nbuf, nchunks)):
        in_copy(k).start()
    for k in range(nchunks):
        in_copy(k).wait()
        for b in range(bsz):
            out_copy(k, b).start()
        j = k - (nbuf - 1)
        if j >= 0:
            for b in range(bsz):
                out_copy(j, b).wait()
            if j + nbuf < nchunks:
                in_copy(j + nbuf).start()
    for j in range(max(0, nchunks - nbuf + 1), nchunks):
        for b in range(bsz):
            out_copy(j, b).wait()


def kernel(input, weights):
    seq_len, bsz = input.shape
    init_size, dim = weights.shape
    info = plsc.get_sparse_core_info()
    nw = info.num_cores * info.num_subcores
    rows_per_w = seq_len // nw
    cb = _CB if rows_per_w % _CB == 0 else rows_per_w
    nchunks = rows_per_w // cb
    nbuf = min(_NBUF, nchunks)
    mesh = plsc.VectorSubcoreMesh(core_axis_name="c", subcore_axis_name="s")
    body = functools.partial(
        _sc_body, bsz=bsz, dim=dim, cb=cb, rows_per_w=rows_per_w,
        nchunks=nchunks, nbuf=nbuf, num_subcores=info.num_subcores)
    return pl.kernel(
        body,
        out_type=jax.ShapeDtypeStruct((seq_len, bsz, dim), weights.dtype),
        mesh=mesh,
        scratch_types=[
            pltpu.VMEM((nbuf, cb, dim), weights.dtype),
            pltpu.SemaphoreType.DMA((nbuf,)),
            pltpu.SemaphoreType.DMA((nbuf, bsz)),
        ],
    )(weights[:seq_len])
